# baseline (speedup 1.0000x reference)
import functools

import jax
import jax.numpy as jnp
from jax import lax
from jax.experimental import pallas as pl
from jax.experimental.pallas import tpu as pltpu
from jax.experimental.pallas import tpu_sc as plsc

NCP = 1024


G = 8
SUB = 8


def _argmax_row(x, kdim):
    m = jnp.max(x)
    io = (lax.broadcasted_iota(jnp.int32, x.shape, 0) * x.shape[1]
          + lax.broadcasted_iota(jnp.int32, x.shape, 1))
    return jnp.min(jnp.where(x == m, io, kdim))


def _fwd_body(fpos_ref, fprev_ref, fpu_ref, fn_ref, *refs):
    i = pl.program_id(0)
    mrows = refs[:G]
    grows = refs[G:2 * G]
    marg_ref = refs[2 * G]
    out_ref = refs[2 * G + 1]
    kdim = mrows[0].shape[-1]
    marg = marg_ref[0]
    for j in range(G):
        @pl.when(i * G + j < fn_ref[0])
        def _():
            pu = fpu_ref[i * G + j]
            p = mrows[j][...].reshape(SUB, kdim // SUB)
            probs = jnp.where(pu > 0, p, marg)
            x = jnp.log(probs + 1e-12) + grows[j][0]
            out_ref[j, :] = jnp.full((128,), _argmax_row(x, kdim), jnp.int32)


def _bwd_body(cpos_ref, cprev_ref, cn_ref, *refs):
    i = pl.program_id(0)
    mrows = refs[:G]
    colblk_ref = refs[G]
    grows = refs[G + 1:2 * G + 1]
    out_ref = refs[2 * G + 1]
    kdim = mrows[0].shape[-1]
    for j in range(G):
        @pl.when(i * G + j < cn_ref[0])
        def _():
            col = colblk_ref[j]
            bs = jnp.sum(col)
            bwdn = col / (bs + 1e-10)
            comb = mrows[j][...].reshape(SUB, kdim // SUB) * bwdn
            cs = jnp.sum(comb)
            combn = comb / (cs + 1e-10)
            x = jnp.log(combn + 1e-12) + grows[j][0]
            out_ref[j, :] = jnp.full((128,), _argmax_row(x, kdim), jnp.int32)


def _sc_gather_cols(mflat, idx_all, K):
    info = plsc.get_sparse_core_info()
    nc_cores, nsub = info.num_cores, info.num_subcores
    nworkers = nc_cores * nsub
    trip = -(-NCP // nworkers)
    mesh = plsc.VectorSubcoreMesh(core_axis_name="c", subcore_axis_name="s")

    @functools.partial(
        pl.kernel, mesh=mesh,
        out_type=jax.ShapeDtypeStruct((NCP * K,), jnp.float32),
        scratch_types=[
            pltpu.VMEM((K,), jnp.int32),
            pltpu.VMEM((K,), jnp.float32),
            pltpu.SemaphoreType.DMA,
        ],
    )
    def k(mflat_hbm, idx_hbm, out_hbm, idx_v, col_v, sem):
        wid = lax.axis_index("s") * nc_cores + lax.axis_index("c")

        def body(it, carry):
            s = jnp.minimum(wid * trip + it, NCP - 1)
            pltpu.sync_copy(idx_hbm.at[pl.ds(s * K, K)], idx_v)
            pltpu.async_copy(mflat_hbm.at[idx_v], col_v, sem).wait()
            pltpu.sync_copy(col_v, out_hbm.at[pl.ds(s * K, K)])
            return carry

        lax.fori_loop(0, trip, body, 0)

    return k(mflat, idx_all)


def kernel(z_masked, mask_indicator, M, marginal):
    T = z_masked.shape[0]
    K = M.shape[1]
    key = jax.random.key(42)
    kf, kb = jax.random.split(key)
    gf = jax.random.gumbel(kf, (T, 8, K // 8), jnp.float32)
    gb = jax.random.gumbel(kb, (T, 8, K // 8), jnp.float32)

    z = z_masked
    prev_sym = jnp.concatenate([jnp.zeros((1,), z.dtype), z[:-1]])
    prev_unmasked = jnp.concatenate(
        [jnp.array([False]), ~mask_indicator[:-1]])
    next_sym = jnp.concatenate([z[1:], jnp.zeros((1,), z.dtype)])
    next_unmasked = jnp.concatenate(
        [~mask_indicator[1:], jnp.array([False])])
    cond = mask_indicator & prev_unmasked & next_unmasked

    fperm = jnp.argsort(~mask_indicator, stable=True).astype(jnp.int32)
    nf = jnp.sum(mask_indicator.astype(jnp.int32))
    valid_f = jnp.arange(T, dtype=jnp.int32) < nf
    fpos = jnp.where(valid_f, fperm, fperm[0])
    f_pu = prev_unmasked[fpos].astype(jnp.int32)
    f_prev = jnp.where(f_pu > 0, prev_sym[fpos], 0).astype(jnp.int32)
    finv = jnp.argsort(fperm, stable=True).astype(jnp.int32)

    cperm_full = jnp.argsort(~cond, stable=True).astype(jnp.int32)
    nc = jnp.sum(cond.astype(jnp.int32))
    cperm = cperm_full[:NCP]
    valid_c = jnp.arange(NCP, dtype=jnp.int32) < nc
    cpos = jnp.where(valid_c, cperm, cperm[0])
    c_prev = prev_sym[cpos].astype(jnp.int32)
    c_cols = next_sym[cpos].astype(jnp.int32)
    cinv = jnp.argsort(cperm_full, stable=True).astype(jnp.int32)

    mflat = M.reshape(K * K)
    iota_k = jnp.arange(K, dtype=jnp.int32)
    idx_all = jnp.where(
        valid_c[:, None],
        c_cols[:, None] + (iota_k * K)[None, :],
        iota_k[None, :]).reshape(-1)
    colrows_flat = _sc_gather_cols(mflat, idx_all, K)

    marg3 = marginal.reshape(1, SUB, K // SUB)
    colrows3 = colrows_flat.reshape(NCP, SUB, K // SUB)

    M3 = M.reshape(K, 1, K)

    def _mrow_spec(j):
        return pl.BlockSpec((1, 1, K),
                            lambda i, pos, prev, *_: (prev[i * G + j], 0, 0))

    def _grow_spec(j):
        return pl.BlockSpec((1, SUB, K // SUB),
                            lambda i, pos, prev, *_: (pos[i * G + j], 0, 0))

    fwd_spec = pltpu.PrefetchScalarGridSpec(
        num_scalar_prefetch=4,
        grid=(T // G,),
        in_specs=([_mrow_spec(j) for j in range(G)]
                  + [_grow_spec(j) for j in range(G)]
                  + [pl.BlockSpec((1, SUB, K // SUB), lambda i, *_: (0, 0, 0))]),
        out_specs=pl.BlockSpec((G, 128), lambda i, *_: (i, 0)),
    )
    samp_f3 = pl.pallas_call(
        _fwd_body,
        grid_spec=fwd_spec,
        out_shape=jax.ShapeDtypeStruct((T, 128), jnp.int32),
    )(fpos, f_prev, f_pu, nf.reshape(1), *([M3] * G), *([gf] * G), marg3)
    samp_f = samp_f3[:, 0]

    def _colblk_map(i, pos, prev, cn):
        return (jnp.minimum(i, jnp.maximum(cn[0] - 1, 0) // G), 0, 0)

    bwd_spec = pltpu.PrefetchScalarGridSpec(
        num_scalar_prefetch=3,
        grid=(NCP // G,),
        in_specs=([_mrow_spec(j) for j in range(G)]
                  + [pl.BlockSpec((G, SUB, K // SUB), _colblk_map)]
                  + [_grow_spec(j) for j in range(G)]),
        out_specs=pl.BlockSpec((G, 128), lambda i, *_: (i, 0)),
    )
    samp_b3 = pl.pallas_call(
        _bwd_body,
        grid_spec=bwd_spec,
        out_shape=jax.ShapeDtypeStruct((NCP, 128), jnp.int32),
    )(cpos, c_prev, nc.reshape(1), *([M3] * G), colrows3, *([gb] * G))
    samp_b = samp_b3[:, 0]

    samp_f_full = jnp.take(samp_f, finv)
    samp_b_full = jnp.take(samp_b, jnp.minimum(cinv, NCP - 1))
    z1 = jnp.where(mask_indicator, samp_f_full, z)
    z_smooth = jnp.where(cond, samp_b_full, z1).astype(z.dtype)
    return z_smooth


# speedup vs baseline: 1.0154x; 1.0154x over previous
import functools

import jax
import jax.numpy as jnp
from jax import lax
from jax.experimental import pallas as pl
from jax.experimental.pallas import tpu as pltpu
from jax.experimental.pallas import tpu_sc as plsc

NCP = 1024


G = 8
SUB = 8


def _argmax_row(x, kdim):
    m = jnp.max(x)
    io = (lax.broadcasted_iota(jnp.int32, x.shape, 0) * x.shape[1]
          + lax.broadcasted_iota(jnp.int32, x.shape, 1))
    return jnp.min(jnp.where(x == m, io, kdim))


def _fwd_body(fpos_ref, fprev_ref, fpu_ref, fn_ref, *refs):
    i = pl.program_id(0)
    mrows = refs[:G]
    grows = refs[G:2 * G]
    marg_ref = refs[2 * G]
    out_ref = refs[2 * G + 1]
    kdim = mrows[0].shape[-1]
    marg = marg_ref[0]
    for j in range(G):
        @pl.when(i * G + j < fn_ref[0])
        def _():
            pu = fpu_ref[i * G + j]
            p = mrows[j][...].reshape(SUB, kdim // SUB)
            probs = jnp.where(pu > 0, p, marg)
            x = jnp.log(probs + 1e-12) + grows[j][0]
            out_ref[j, :] = jnp.full((128,), _argmax_row(x, kdim), jnp.int32)


def _bwd_body(cpos_ref, cprev_ref, cn_ref, *refs):
    i = pl.program_id(0)
    mrows = refs[:G]
    colblk_ref = refs[G]
    grows = refs[G + 1:2 * G + 1]
    out_ref = refs[2 * G + 1]
    kdim = mrows[0].shape[-1]
    for j in range(G):
        @pl.when(i * G + j < cn_ref[0])
        def _():
            col = colblk_ref[j]
            bs = jnp.sum(col)
            bwdn = col / (bs + 1e-10)
            comb = mrows[j][...].reshape(SUB, kdim // SUB) * bwdn
            cs = jnp.sum(comb)
            combn = comb / (cs + 1e-10)
            x = jnp.log(combn + 1e-12) + grows[j][0]
            out_ref[j, :] = jnp.full((128,), _argmax_row(x, kdim), jnp.int32)


def _sc_gather_cols(mflat, idx_all, K):
    info = plsc.get_sparse_core_info()
    nc_cores, nsub = info.num_cores, info.num_subcores
    nworkers = nc_cores * nsub
    trip = -(-NCP // nworkers)
    mesh = plsc.VectorSubcoreMesh(core_axis_name="c", subcore_axis_name="s")

    @functools.partial(
        pl.kernel, mesh=mesh,
        out_type=jax.ShapeDtypeStruct((NCP * K,), jnp.float32),
        scratch_types=[
            pltpu.VMEM((K,), jnp.int32),
            pltpu.VMEM((K,), jnp.float32),
            pltpu.SemaphoreType.DMA,
        ],
    )
    def k(mflat_hbm, idx_hbm, out_hbm, idx_v, col_v, sem):
        wid = lax.axis_index("s") * nc_cores + lax.axis_index("c")

        def body(it, carry):
            s = jnp.minimum(wid * trip + it, NCP - 1)
            pltpu.sync_copy(idx_hbm.at[pl.ds(s * K, K)], idx_v)
            pltpu.async_copy(mflat_hbm.at[idx_v], col_v, sem).wait()
            pltpu.sync_copy(col_v, out_hbm.at[pl.ds(s * K, K)])
            return carry

        lax.fori_loop(0, trip, body, 0)

    return k(mflat, idx_all)


def kernel(z_masked, mask_indicator, M, marginal):
    T = z_masked.shape[0]
    K = M.shape[1]
    key = jax.random.key(42)
    kf, kb = jax.random.split(key)
    gf = jax.random.gumbel(kf, (T, 8, K // 8), jnp.float32)
    gb = jax.random.gumbel(kb, (T, 8, K // 8), jnp.float32)

    z = z_masked
    prev_sym = jnp.concatenate([jnp.zeros((1,), z.dtype), z[:-1]])
    prev_unmasked = jnp.concatenate(
        [jnp.array([False]), ~mask_indicator[:-1]])
    next_sym = jnp.concatenate([z[1:], jnp.zeros((1,), z.dtype)])
    next_unmasked = jnp.concatenate(
        [~mask_indicator[1:], jnp.array([False])])
    cond = mask_indicator & prev_unmasked & next_unmasked

    fperm = jnp.argsort(~mask_indicator, stable=True).astype(jnp.int32)
    nf = jnp.sum(mask_indicator.astype(jnp.int32))
    valid_f = jnp.arange(T, dtype=jnp.int32) < nf
    fpos = jnp.where(valid_f, fperm, fperm[0])
    f_pu = prev_unmasked[fpos].astype(jnp.int32)
    f_prev = jnp.where(f_pu > 0, prev_sym[fpos], 0).astype(jnp.int32)
    finv = jnp.argsort(fperm, stable=True).astype(jnp.int32)

    cperm_full = jnp.argsort(~cond, stable=True).astype(jnp.int32)
    nc = jnp.sum(cond.astype(jnp.int32))
    cperm = cperm_full[:NCP]
    valid_c = jnp.arange(NCP, dtype=jnp.int32) < nc
    cpos = jnp.where(valid_c, cperm, cperm[0])
    c_prev = prev_sym[cpos].astype(jnp.int32)
    c_cols = next_sym[cpos].astype(jnp.int32)
    cinv = jnp.argsort(cperm_full, stable=True).astype(jnp.int32)

    mflat = M.reshape(K * K)
    idx_all = (c_cols[:, None]
               + (jnp.arange(K, dtype=jnp.int32) * K)[None, :]).reshape(-1)
    colrows_flat = _sc_gather_cols(mflat, idx_all, K)

    marg3 = marginal.reshape(1, SUB, K // SUB)
    colrows3 = colrows_flat.reshape(NCP, SUB, K // SUB)

    M3 = M.reshape(K, 1, K)

    def _mrow_spec(j):
        return pl.BlockSpec((1, 1, K),
                            lambda i, pos, prev, *_: (prev[i * G + j], 0, 0))

    def _grow_spec(j):
        return pl.BlockSpec((1, SUB, K // SUB),
                            lambda i, pos, prev, *_: (pos[i * G + j], 0, 0))

    fwd_spec = pltpu.PrefetchScalarGridSpec(
        num_scalar_prefetch=4,
        grid=(T // G,),
        in_specs=([_mrow_spec(j) for j in range(G)]
                  + [_grow_spec(j) for j in range(G)]
                  + [pl.BlockSpec((1, SUB, K // SUB), lambda i, *_: (0, 0, 0))]),
        out_specs=pl.BlockSpec((G, 128), lambda i, *_: (i, 0)),
    )
    samp_f3 = pl.pallas_call(
        _fwd_body,
        grid_spec=fwd_spec,
        out_shape=jax.ShapeDtypeStruct((T, 128), jnp.int32),
    )(fpos, f_prev, f_pu, nf.reshape(1), *([M3] * G), *([gf] * G), marg3)
    samp_f = samp_f3[:, 0]

    def _colblk_map(i, pos, prev, cn):
        return (jnp.minimum(i, jnp.maximum(cn[0] - 1, 0) // G), 0, 0)

    bwd_spec = pltpu.PrefetchScalarGridSpec(
        num_scalar_prefetch=3,
        grid=(NCP // G,),
        in_specs=([_mrow_spec(j) for j in range(G)]
                  + [pl.BlockSpec((G, SUB, K // SUB), _colblk_map)]
                  + [_grow_spec(j) for j in range(G)]),
        out_specs=pl.BlockSpec((G, 128), lambda i, *_: (i, 0)),
    )
    samp_b3 = pl.pallas_call(
        _bwd_body,
        grid_spec=bwd_spec,
        out_shape=jax.ShapeDtypeStruct((NCP, 128), jnp.int32),
    )(cpos, c_prev, nc.reshape(1), *([M3] * G), colrows3, *([gb] * G))
    samp_b = samp_b3[:, 0]

    samp_f_full = jnp.take(samp_f, finv)
    samp_b_full = jnp.take(samp_b, jnp.minimum(cinv, NCP - 1))
    z1 = jnp.where(mask_indicator, samp_f_full, z)
    z_smooth = jnp.where(cond, samp_b_full, z1).astype(z.dtype)
    return z_smooth


# speedup vs baseline: 1.0155x; 1.0001x over previous
import functools

import jax
import jax.numpy as jnp
from jax import lax
from jax.experimental import pallas as pl
from jax.experimental.pallas import tpu as pltpu
from jax.experimental.pallas import tpu_sc as plsc

NCP = 1024


G = 16
SUB = 8


def _argmax_row(x, kdim):
    m = jnp.max(x)
    io = (lax.broadcasted_iota(jnp.int32, x.shape, 0) * x.shape[1]
          + lax.broadcasted_iota(jnp.int32, x.shape, 1))
    return jnp.min(jnp.where(x == m, io, kdim))


def _fwd_body(fpos_ref, fprev_ref, fpu_ref, fn_ref, *refs):
    i = pl.program_id(0)
    mrows = refs[:G]
    grows = refs[G:2 * G]
    marg_ref = refs[2 * G]
    out_ref = refs[2 * G + 1]
    kdim = mrows[0].shape[-1]
    marg = marg_ref[0]
    for j in range(G):
        @pl.when(i * G + j < fn_ref[0])
        def _():
            pu = fpu_ref[i * G + j]
            p = mrows[j][...].reshape(SUB, kdim // SUB)
            probs = jnp.where(pu > 0, p, marg)
            x = jnp.log(probs + 1e-12) + grows[j][0]
            out_ref[j, :] = jnp.full((128,), _argmax_row(x, kdim), jnp.int32)


def _bwd_body(cpos_ref, cprev_ref, cn_ref, *refs):
    i = pl.program_id(0)
    mrows = refs[:G]
    colblk_ref = refs[G]
    grows = refs[G + 1:2 * G + 1]
    out_ref = refs[2 * G + 1]
    kdim = mrows[0].shape[-1]
    for j in range(G):
        @pl.when(i * G + j < cn_ref[0])
        def _():
            col = colblk_ref[j]
            bs = jnp.sum(col)
            bwdn = col / (bs + 1e-10)
            comb = mrows[j][...].reshape(SUB, kdim // SUB) * bwdn
            cs = jnp.sum(comb)
            combn = comb / (cs + 1e-10)
            x = jnp.log(combn + 1e-12) + grows[j][0]
            out_ref[j, :] = jnp.full((128,), _argmax_row(x, kdim), jnp.int32)


def _sc_gather_cols(mflat, idx_all, K):
    info = plsc.get_sparse_core_info()
    nc_cores, nsub = info.num_cores, info.num_subcores
    nworkers = nc_cores * nsub
    trip = -(-NCP // nworkers)
    mesh = plsc.VectorSubcoreMesh(core_axis_name="c", subcore_axis_name="s")

    @functools.partial(
        pl.kernel, mesh=mesh,
        out_type=jax.ShapeDtypeStruct((NCP * K,), jnp.float32),
        scratch_types=[
            pltpu.VMEM((K,), jnp.int32),
            pltpu.VMEM((K,), jnp.float32),
            pltpu.SemaphoreType.DMA,
        ],
    )
    def k(mflat_hbm, idx_hbm, out_hbm, idx_v, col_v, sem):
        wid = lax.axis_index("s") * nc_cores + lax.axis_index("c")

        def body(it, carry):
            s = jnp.minimum(wid * trip + it, NCP - 1)
            pltpu.sync_copy(idx_hbm.at[pl.ds(s * K, K)], idx_v)
            pltpu.async_copy(mflat_hbm.at[idx_v], col_v, sem).wait()
            pltpu.sync_copy(col_v, out_hbm.at[pl.ds(s * K, K)])
            return carry

        lax.fori_loop(0, trip, body, 0)

    return k(mflat, idx_all)


def kernel(z_masked, mask_indicator, M, marginal):
    T = z_masked.shape[0]
    K = M.shape[1]
    key = jax.random.key(42)
    kf, kb = jax.random.split(key)
    gf = jax.random.gumbel(kf, (T, 8, K // 8), jnp.float32)
    gb = jax.random.gumbel(kb, (T, 8, K // 8), jnp.float32)

    z = z_masked
    prev_sym = jnp.concatenate([jnp.zeros((1,), z.dtype), z[:-1]])
    prev_unmasked = jnp.concatenate(
        [jnp.array([False]), ~mask_indicator[:-1]])
    next_sym = jnp.concatenate([z[1:], jnp.zeros((1,), z.dtype)])
    next_unmasked = jnp.concatenate(
        [~mask_indicator[1:], jnp.array([False])])
    cond = mask_indicator & prev_unmasked & next_unmasked

    fperm = jnp.argsort(~mask_indicator, stable=True).astype(jnp.int32)
    nf = jnp.sum(mask_indicator.astype(jnp.int32))
    valid_f = jnp.arange(T, dtype=jnp.int32) < nf
    fpos = jnp.where(valid_f, fperm, fperm[0])
    f_pu = prev_unmasked[fpos].astype(jnp.int32)
    f_prev = jnp.where(f_pu > 0, prev_sym[fpos], 0).astype(jnp.int32)
    finv = jnp.argsort(fperm, stable=True).astype(jnp.int32)

    cperm_full = jnp.argsort(~cond, stable=True).astype(jnp.int32)
    nc = jnp.sum(cond.astype(jnp.int32))
    cperm = cperm_full[:NCP]
    valid_c = jnp.arange(NCP, dtype=jnp.int32) < nc
    cpos = jnp.where(valid_c, cperm, cperm[0])
    c_prev = prev_sym[cpos].astype(jnp.int32)
    c_cols = next_sym[cpos].astype(jnp.int32)
    cinv = jnp.argsort(cperm_full, stable=True).astype(jnp.int32)

    mflat = M.reshape(K * K)
    idx_all = (c_cols[:, None]
               + (jnp.arange(K, dtype=jnp.int32) * K)[None, :]).reshape(-1)
    colrows_flat = _sc_gather_cols(mflat, idx_all, K)

    marg3 = marginal.reshape(1, SUB, K // SUB)
    colrows3 = colrows_flat.reshape(NCP, SUB, K // SUB)

    M3 = M.reshape(K, 1, K)

    def _mrow_spec(j):
        return pl.BlockSpec((1, 1, K),
                            lambda i, pos, prev, *_: (prev[i * G + j], 0, 0))

    def _grow_spec(j):
        return pl.BlockSpec((1, SUB, K // SUB),
                            lambda i, pos, prev, *_: (pos[i * G + j], 0, 0))

    fwd_spec = pltpu.PrefetchScalarGridSpec(
        num_scalar_prefetch=4,
        grid=(T // G,),
        in_specs=([_mrow_spec(j) for j in range(G)]
                  + [_grow_spec(j) for j in range(G)]
                  + [pl.BlockSpec((1, SUB, K // SUB), lambda i, *_: (0, 0, 0))]),
        out_specs=pl.BlockSpec((G, 128), lambda i, *_: (i, 0)),
    )
    samp_f3 = pl.pallas_call(
        _fwd_body,
        grid_spec=fwd_spec,
        out_shape=jax.ShapeDtypeStruct((T, 128), jnp.int32),
    )(fpos, f_prev, f_pu, nf.reshape(1), *([M3] * G), *([gf] * G), marg3)
    samp_f = samp_f3[:, 0]

    def _colblk_map(i, pos, prev, cn):
        return (jnp.minimum(i, jnp.maximum(cn[0] - 1, 0) // G), 0, 0)

    bwd_spec = pltpu.PrefetchScalarGridSpec(
        num_scalar_prefetch=3,
        grid=(NCP // G,),
        in_specs=([_mrow_spec(j) for j in range(G)]
                  + [pl.BlockSpec((G, SUB, K // SUB), _colblk_map)]
                  + [_grow_spec(j) for j in range(G)]),
        out_specs=pl.BlockSpec((G, 128), lambda i, *_: (i, 0)),
    )
    samp_b3 = pl.pallas_call(
        _bwd_body,
        grid_spec=bwd_spec,
        out_shape=jax.ShapeDtypeStruct((NCP, 128), jnp.int32),
    )(cpos, c_prev, nc.reshape(1), *([M3] * G), colrows3, *([gb] * G))
    samp_b = samp_b3[:, 0]

    samp_f_full = jnp.take(samp_f, finv)
    samp_b_full = jnp.take(samp_b, jnp.minimum(cinv, NCP - 1))
    z1 = jnp.where(mask_indicator, samp_f_full, z)
    z_smooth = jnp.where(cond, samp_b_full, z1).astype(z.dtype)
    return z_smooth
